# Initial kernel scaffold; baseline (speedup 1.0000x reference)
import jax
import jax.numpy as jnp
from jax.experimental import pallas as pl


def kernel(x, edge_index, edge_weight, W, b):
    raise NotImplementedError("write your pallas kernel here")


# speedup vs baseline: 5.0554x; 5.0554x over previous
import functools

import jax
import jax.numpy as jnp
from jax import lax
from jax.experimental import pallas as pl
from jax.experimental.pallas import tpu as pltpu
from jax.experimental.pallas import tpu_sc as plsc

N = 10000
E = 320000
D = 128

NC = 2
NS = 16
NW = NC * NS

EPT = E // NW
CH = 128
NFULL = EPT // CH
TAIL = EPT - NFULL * CH
RPT = N // NS

_mesh = plsc.VectorSubcoreMesh(core_axis_name="c", subcore_axis_name="s")


@functools.partial(
    pl.kernel,
    mesh=_mesh,
    out_type=jax.ShapeDtypeStruct((NC, N, D), jnp.float32),
    scratch_types=[
        pltpu.VMEM((CH,), jnp.int32),
        pltpu.VMEM((CH,), jnp.int32),
        pltpu.VMEM((CH,), jnp.float32),
        pltpu.VMEM((CH, D), jnp.float32),
        pltpu.VMEM((TAIL,), jnp.int32),
        pltpu.VMEM((TAIL,), jnp.int32),
        pltpu.VMEM((TAIL,), jnp.float32),
        pltpu.VMEM((TAIL, D), jnp.float32),
        pltpu.VMEM_SHARED((N, D), jnp.float32),
        pltpu.SemaphoreType.DMA,
    ],
)
def _spmm(x_hbm, row_hbm, col_hbm, w_hbm, out_hbm,
          colv, rowv, wv, rows, colt, rowt, wt, rowst, agg, sem):
    c = lax.axis_index("c")
    s = lax.axis_index("s")
    wid = c * NS + s

    def _zero(i, _):
        for j in range(8):
            rows[i, pl.ds(j * 16, 16)] = jnp.zeros((16,), jnp.float32)
        return 0
    lax.fori_loop(0, CH, _zero, 0)

    NRC = N // CH
    RTAIL = N - NRC * CH
    for k in range(NRC // NS + 1):
        q = s + NS * k
        @pl.when(q < NRC)
        def _():
            pltpu.sync_copy(rows, agg.at[pl.ds(q * CH, CH)])
    @pl.when(s == 0)
    def _():
        pltpu.sync_copy(rows.at[pl.ds(0, RTAIL)],
                        agg.at[pl.ds(NRC * CH, RTAIL)])
    plsc.subcore_barrier()

    ebase = wid * EPT

    def _scale(wref, rref, sz):
        def body(g, _):
            w16 = wref[pl.ds(g * 16, 16)]
            for j in range(16):
                wj = w16[j]
                e = g * 16 + j
                for k in range(8):
                    rref[e, pl.ds(k * 16, 16)] = rref[e, pl.ds(k * 16, 16)] * wj
            return 0
        lax.fori_loop(0, sz // 16, body, 0)

    def _chunk(cc, _):
        off = pl.multiple_of(ebase + cc * CH, 8)
        pltpu.sync_copy(col_hbm.at[pl.ds(off, CH)], colv)
        pltpu.sync_copy(row_hbm.at[pl.ds(off, CH)], rowv)
        pltpu.sync_copy(w_hbm.at[pl.ds(off, CH)], wv)
        pltpu.async_copy(x_hbm.at[colv], rows, sem).wait()
        _scale(wv, rows, CH)
        pltpu.sync_copy(rows, agg.at[rowv], add=True)
        return 0
    lax.fori_loop(0, NFULL, _chunk, 0)

    toff = pl.multiple_of(ebase + NFULL * CH, 8)
    pltpu.sync_copy(col_hbm.at[pl.ds(toff, TAIL)], colt)
    pltpu.sync_copy(row_hbm.at[pl.ds(toff, TAIL)], rowt)
    pltpu.sync_copy(w_hbm.at[pl.ds(toff, TAIL)], wt)
    pltpu.async_copy(x_hbm.at[colt], rowst, sem).wait()
    _scale(wt, rowst, TAIL)
    pltpu.sync_copy(rowst, agg.at[rowt], add=True)

    plsc.subcore_barrier()

    for k in range(NRC // NS + 1):
        q = s + NS * k
        @pl.when(q < NRC)
        def _():
            pltpu.sync_copy(agg.at[pl.ds(q * CH, CH)], rows)
            pltpu.sync_copy(rows, out_hbm.at[c, pl.ds(q * CH, CH)])
    @pl.when(s == 0)
    def _():
        pltpu.sync_copy(agg.at[pl.ds(NRC * CH, RTAIL)], rowst)
        pltpu.sync_copy(rowst, out_hbm.at[c, pl.ds(NRC * CH, RTAIL)])


def _tc_body(p_ref, wt_ref, b_ref, o_ref):
    ssum = p_ref[0] + p_ref[1]
    o_ref[...] = jnp.dot(ssum, wt_ref[...],
                         preferred_element_type=jnp.float32,
                         precision=lax.Precision.HIGHEST) + b_ref[...]


_linear = pl.pallas_call(
    _tc_body,
    grid=(10,),
    in_specs=[
        pl.BlockSpec((NC, N // 10, D), lambda i: (0, i, 0)),
        pl.BlockSpec((D, D), lambda i: (0, 0)),
        pl.BlockSpec((1, D), lambda i: (0, 0)),
    ],
    out_specs=pl.BlockSpec((N // 10, D), lambda i: (i, 0)),
    out_shape=jax.ShapeDtypeStruct((N, D), jnp.float32),
)


def kernel(x, edge_index, edge_weight, W, b):
    row = edge_index[0].astype(jnp.int32)
    col = edge_index[1].astype(jnp.int32)
    partials = _spmm(x, row, col, edge_weight)
    return _linear(partials, W.T, b[None, :])


# speedup vs baseline: 10.9818x; 2.1723x over previous
import functools

import jax
import jax.numpy as jnp
from jax import lax
from jax.experimental import pallas as pl
from jax.experimental.pallas import tpu as pltpu
from jax.experimental.pallas import tpu_sc as plsc

N = 10000
E = 320000
D = 128

NC = 2
NS = 16
NW = NC * NS

EPT = E // NW
CH = 128
NFULL = EPT // CH
TAIL = EPT - NFULL * CH
NB = 3

_mesh = plsc.VectorSubcoreMesh(core_axis_name="c", subcore_axis_name="s")


@functools.partial(
    pl.kernel,
    mesh=_mesh,
    out_type=jax.ShapeDtypeStruct((NC, N, D), jnp.float32),
    scratch_types=[
        pltpu.VMEM((CH, D), jnp.float32),
        pltpu.VMEM((CH, D), jnp.float32),
        pltpu.VMEM((CH, D), jnp.float32),
        pltpu.VMEM((CH,), jnp.int32),
        pltpu.VMEM((CH,), jnp.int32),
        pltpu.VMEM((CH,), jnp.int32),
        pltpu.VMEM((CH,), jnp.int32),
        pltpu.VMEM((CH,), jnp.int32),
        pltpu.VMEM((CH,), jnp.int32),
        pltpu.VMEM((CH,), jnp.float32),
        pltpu.VMEM((CH,), jnp.float32),
        pltpu.VMEM((CH,), jnp.float32),
        pltpu.VMEM((TAIL,), jnp.int32),
        pltpu.VMEM((TAIL,), jnp.int32),
        pltpu.VMEM_SHARED((N, D), jnp.float32),
        pltpu.SemaphoreType.DMA,
        pltpu.SemaphoreType.DMA,
        pltpu.SemaphoreType.DMA,
        pltpu.SemaphoreType.DMA,
        pltpu.SemaphoreType.DMA,
        pltpu.SemaphoreType.DMA,
        pltpu.SemaphoreType.DMA,
        pltpu.SemaphoreType.DMA,
        pltpu.SemaphoreType.DMA,
        pltpu.SemaphoreType.DMA,
    ],
)
def _spmm(x_hbm, row_hbm, col_hbm, w_hbm, out_hbm,
          r0, r1, r2, c0, c1, c2, i0, i1, i2, w0, w1, w2,
          colt, rowt, agg,
          sg0, sg1, sg2, ss0, ss1, ss2, si0, si1, si2, sem):
    c = lax.axis_index("c")
    s = lax.axis_index("s")
    wid = c * NS + s
    ebase = pl.multiple_of(wid * EPT, 8)

    rows = (r0, r1, r2)
    colv = (c0, c1, c2)
    ridx = (i0, i1, i2)
    wv = (w0, w1, w2)
    sg = (sg0, sg1, sg2)
    ss = (ss0, ss1, ss2)
    si = (si0, si1, si2)

    def _zero(i, _):
        for j in range(8):
            r0[i, pl.ds(j * 16, 16)] = jnp.zeros((16,), jnp.float32)
        return 0
    lax.fori_loop(0, CH, _zero, 0)

    NRC = N // CH
    RTAIL = N - NRC * CH
    for k in range(NRC // NS + 1):
        q = s + NS * k
        @pl.when(q < NRC)
        def _():
            pltpu.sync_copy(r0, agg.at[pl.ds(q * CH, CH)])
    @pl.when(s == 0)
    def _():
        pltpu.sync_copy(r0.at[pl.ds(0, RTAIL)], agg.at[pl.ds(NRC * CH, RTAIL)])
    plsc.subcore_barrier()

    def _stage(cc, b):
        off = pl.multiple_of(ebase + cc * CH, 8)
        pltpu.async_copy(col_hbm.at[pl.ds(off, CH)], colv[b], si[b])
        pltpu.async_copy(row_hbm.at[pl.ds(off, CH)], ridx[b], si[b])
        pltpu.async_copy(w_hbm.at[pl.ds(off, CH)], wv[b], si[b])

    def _wait_stage(b):
        pltpu.make_async_copy(col_hbm.at[pl.ds(0, CH)], colv[b], si[b]).wait()
        pltpu.make_async_copy(row_hbm.at[pl.ds(0, CH)], ridx[b], si[b]).wait()
        pltpu.make_async_copy(w_hbm.at[pl.ds(0, CH)], wv[b], si[b]).wait()

    def _wait_scatter(b):
        pltpu.make_async_copy(rows[b], agg.at[ridx[b]], ss[b]).wait()

    def _scale(b, sz):
        def body(g, _):
            w16 = wv[b][pl.ds(g * 16, 16)]
            for j in range(16):
                wj = w16[j]
                e = g * 16 + j
                for k in range(8):
                    rows[b][e, pl.ds(k * 16, 16)] = (
                        rows[b][e, pl.ds(k * 16, 16)] * wj)
            return 0
        lax.fori_loop(0, sz // 16, body, 0)

    _stage(0, 0)
    _stage(1, 1)
    _wait_stage(0)
    pltpu.async_copy(x_hbm.at[colv[0]], rows[0], sg[0])

    def _visit(v, b):
        bn = (b + 1) % NB
        bs = (b + 2) % NB
        @pl.when(v + 1 < NFULL)
        def _():
            _wait_stage(bn)
            pltpu.async_copy(x_hbm.at[colv[bn]], rows[bn], sg[bn])
        @pl.when(v >= 1)
        def _():
            _wait_scatter(bs)
        @pl.when(v + 2 < NFULL)
        def _():
            _stage(v + 2, bs)
        pltpu.make_async_copy(x_hbm.at[colv[b]], rows[b], sg[b]).wait()
        _scale(b, CH)
        pltpu.async_copy(rows[b], agg.at[ridx[b]], ss[b], add=True)

    def _triple(g, _):
        for j in range(NB):
            _visit(g * NB + j, j)
        return 0
    lax.fori_loop(0, NFULL // NB, _triple, 0)

    _wait_scatter((NFULL - 1) % NB)

    toff = pl.multiple_of(ebase + NFULL * CH, 8)
    pltpu.sync_copy(col_hbm.at[pl.ds(toff, TAIL)], colt)
    pltpu.sync_copy(row_hbm.at[pl.ds(toff, TAIL)], rowt)
    pltpu.sync_copy(w_hbm.at[pl.ds(toff, TAIL)], wv[0].at[pl.ds(0, TAIL)])
    pltpu.async_copy(x_hbm.at[colt], rows[0].at[pl.ds(0, TAIL)], sem).wait()
    w16 = wv[0][pl.ds(0, TAIL)]
    for j in range(TAIL):
        wj = w16[j]
        for k in range(8):
            rows[0][j, pl.ds(k * 16, 16)] = rows[0][j, pl.ds(k * 16, 16)] * wj
    pltpu.sync_copy(rows[0].at[pl.ds(0, TAIL)], agg.at[rowt], add=True)

    plsc.subcore_barrier()

    for k in range(NRC // NS + 1):
        q = s + NS * k
        @pl.when(q < NRC)
        def _():
            pltpu.sync_copy(agg.at[pl.ds(q * CH, CH)], r0)
            pltpu.sync_copy(r0, out_hbm.at[c, pl.ds(q * CH, CH)])
    @pl.when(s == 0)
    def _():
        pltpu.sync_copy(agg.at[pl.ds(NRC * CH, RTAIL)], r1.at[pl.ds(0, RTAIL)])
        pltpu.sync_copy(r1.at[pl.ds(0, RTAIL)], out_hbm.at[c, pl.ds(NRC * CH, RTAIL)])


def _tc_body(p_ref, wt_ref, b_ref, o_ref):
    ssum = p_ref[0] + p_ref[1]
    o_ref[...] = jnp.dot(ssum, wt_ref[...],
                         preferred_element_type=jnp.float32,
                         precision=lax.Precision.HIGHEST) + b_ref[...]


_linear = pl.pallas_call(
    _tc_body,
    grid=(10,),
    in_specs=[
        pl.BlockSpec((NC, N // 10, D), lambda i: (0, i, 0)),
        pl.BlockSpec((D, D), lambda i: (0, 0)),
        pl.BlockSpec((1, D), lambda i: (0, 0)),
    ],
    out_specs=pl.BlockSpec((N // 10, D), lambda i: (i, 0)),
    out_shape=jax.ShapeDtypeStruct((N, D), jnp.float32),
)


def kernel(x, edge_index, edge_weight, W, b):
    row = edge_index[0].astype(jnp.int32)
    col = edge_index[1].astype(jnp.int32)
    partials = _spmm(x, row, col, edge_weight)
    return _linear(partials, W.T, b[None, :])
